# Initial kernel scaffold; baseline (speedup 1.0000x reference)
import jax
import jax.numpy as jnp
from jax.experimental import pallas as pl


def kernel(z, edge_index):
    raise NotImplementedError("write your pallas kernel here")


# speedup vs baseline: 4.8075x; 4.8075x over previous
import functools

import jax
import jax.numpy as jnp
from jax import lax
from jax.experimental import pallas as pl
from jax.experimental.pallas import tpu as pltpu
from jax.experimental.pallas import tpu_sc as plsc

EPS = 1e-6
N_NODES = 10000
D_FEAT = 128
N_EDGES = 320000

NUM_CORES = 2
NUM_SUBCORES = 16
NUM_WORKERS = NUM_CORES * NUM_SUBCORES
EDGES_PER_WORKER = N_EDGES // NUM_WORKERS
CHUNK = 80
NUM_CHUNKS = EDGES_PER_WORKER // CHUNK
NUM_PAIRS = (NUM_CHUNKS + 1) // 2
LANES = 16
NSUB = D_FEAT // LANES
EDGE_UNROLL = 4


def _tree_sum(vs):
    n = len(vs)
    if n == 1:
        return vs[0]
    return _tree_sum([vs[2 * i] + vs[2 * i + 1] for i in range(n // 2)])


def _vec_rsqrt(t):
    y = plsc.bitcast(
        jnp.int32(0x5F3759DF)
        - lax.shift_right_logical(plsc.bitcast(t, jnp.int32), jnp.int32(1)),
        jnp.float32,
    )
    for _ in range(3):
        y = y * (1.5 - 0.5 * t * y * y)
    return y


def _cosine_body(
    z_hbm, src_hbm, dst_hbm, out_hbm,
    si_v, di_v, o_v, a0, b0, a1, b1, srd, sra, srb,
    sem_a0, sem_b0, sem_a1, sem_b1,
):
    wid = lax.axis_index("s") * jnp.int32(NUM_CORES) + lax.axis_index("c")
    base_w = wid * jnp.int32(EDGES_PER_WORKER)
    pltpu.sync_copy(src_hbm.at[pl.ds(base_w, EDGES_PER_WORKER)], si_v)
    pltpu.sync_copy(dst_hbm.at[pl.ds(base_w, EDGES_PER_WORKER)], di_v)

    abuf = (a0, a1)
    bbuf = (b0, b1)
    asem = (sem_a0, sem_a1)
    bsem = (sem_b0, sem_b1)

    def issue(ci, p):
        off = ci * jnp.int32(CHUNK)
        pltpu.async_copy(z_hbm.at[si_v.at[pl.ds(off, CHUNK)]], abuf[p], asem[p])
        pltpu.async_copy(z_hbm.at[di_v.at[pl.ds(off, CHUNK)]], bbuf[p], bsem[p])

    def wait(p):
        pltpu.make_async_copy(z_hbm.at[si_v.at[pl.ds(0, CHUNK)]], abuf[p], asem[p]).wait()
        pltpu.make_async_copy(z_hbm.at[di_v.at[pl.ds(0, CHUNK)]], bbuf[p], bsem[p]).wait()

    def compute(ci, p):
        av, bv = abuf[p], bbuf[p]
        obase = ci * jnp.int32(CHUNK)

        def group(g, carry):
            ebase = g * jnp.int32(LANES)
            for j in range(LANES):
                e = ebase + jnp.int32(j)
                va = [av[e, pl.ds(LANES * k, LANES)] for k in range(NSUB)]
                vb = [bv[e, pl.ds(LANES * k, LANES)] for k in range(NSUB)]
                d = _tree_sum([va[k] * vb[k] for k in range(NSUB)])
                qa = _tree_sum([va[k] * va[k] for k in range(NSUB)])
                qb = _tree_sum([vb[k] * vb[k] for k in range(NSUB)])
                srd[j, pl.ds(0, LANES)] = plsc.cumsum(d)
                sra[j, pl.ds(0, LANES)] = plsc.cumsum(qa)
                srb[j, pl.ds(0, LANES)] = plsc.cumsum(qb)
            rows = lax.iota(jnp.int32, LANES)
            lastc = jnp.full((LANES,), LANES - 1, jnp.int32)
            dot = plsc.load_gather(srd, [rows, lastc])
            sa = plsc.load_gather(sra, [rows, lastc])
            sb = plsc.load_gather(srb, [rows, lastc])
            t = sa * sb
            s = t * _vec_rsqrt(t)
            o_v[pl.ds(obase + ebase, LANES)] = dot / jnp.maximum(s, EPS)
            return carry

        lax.fori_loop(jnp.int32(0), jnp.int32(CHUNK // LANES), group, jnp.int32(0))

    last = jnp.int32(NUM_CHUNKS - 1)
    issue(jnp.int32(0), 0)

    def pair(pi, carry):
        c0 = pi * jnp.int32(2)
        c1 = jnp.minimum(c0 + 1, last)
        c2 = jnp.minimum(c0 + 2, last)
        issue(c1, 1)
        wait(0)
        compute(c0, 0)
        issue(c2, 0)
        wait(1)
        compute(c1, 1)
        return carry

    lax.fori_loop(jnp.int32(0), jnp.int32(NUM_PAIRS), pair, jnp.int32(0))
    wait(0)
    pltpu.sync_copy(o_v, out_hbm.at[pl.ds(base_w, EDGES_PER_WORKER)])


_cosine_sc = functools.partial(
    pl.kernel,
    out_type=jax.ShapeDtypeStruct((N_EDGES,), jnp.float32),
    mesh=plsc.VectorSubcoreMesh(core_axis_name="c", subcore_axis_name="s"),
    compiler_params=pltpu.CompilerParams(needs_layout_passes=False),
    scratch_types=[
        pltpu.VMEM((EDGES_PER_WORKER,), jnp.int32),
        pltpu.VMEM((EDGES_PER_WORKER,), jnp.int32),
        pltpu.VMEM((EDGES_PER_WORKER,), jnp.float32),
        pltpu.VMEM((CHUNK, D_FEAT), jnp.float32),
        pltpu.VMEM((CHUNK, D_FEAT), jnp.float32),
        pltpu.VMEM((CHUNK, D_FEAT), jnp.float32),
        pltpu.VMEM((CHUNK, D_FEAT), jnp.float32),
        pltpu.VMEM((LANES, LANES + 1), jnp.float32),
        pltpu.VMEM((LANES, LANES + 1), jnp.float32),
        pltpu.VMEM((LANES, LANES + 1), jnp.float32),
        pltpu.SemaphoreType.DMA,
        pltpu.SemaphoreType.DMA,
        pltpu.SemaphoreType.DMA,
        pltpu.SemaphoreType.DMA,
    ],
)(_cosine_body)


def kernel(z, edge_index):
    src = edge_index[0].astype(jnp.int32)
    dst = edge_index[1].astype(jnp.int32)
    return _cosine_sc(z, src, dst)


# speedup vs baseline: 8.7916x; 1.8287x over previous
import functools

import jax
import jax.numpy as jnp
from jax import lax
from jax.experimental import pallas as pl
from jax.experimental.pallas import tpu as pltpu
from jax.experimental.pallas import tpu_sc as plsc

EPS = 1e-6
N_NODES = 10000
D_FEAT = 128
N_EDGES = 320000

NUM_CORES = 2
NUM_SUBCORES = 16
NUM_WORKERS = NUM_CORES * NUM_SUBCORES
EDGES_PER_WORKER = N_EDGES // NUM_WORKERS
CHUNK = 80
NUM_CHUNKS = EDGES_PER_WORKER // CHUNK
NUM_PAIRS = (NUM_CHUNKS + 1) // 2
LANES = 16
NSUB = D_FEAT // LANES

N_PAD = 10240
NODES_PER_TILE = N_PAD // NUM_SUBCORES
NORM_CHUNKS = NODES_PER_TILE // CHUNK


def _tree_sum(vs):
    n = len(vs)
    if n == 1:
        return vs[0]
    return _tree_sum([vs[2 * i] + vs[2 * i + 1] for i in range(n // 2)])


def _vec_rsqrt(t):
    y = plsc.bitcast(
        jnp.int32(0x5F3759DF)
        - lax.shift_right_logical(plsc.bitcast(t, jnp.int32), jnp.int32(1)),
        jnp.float32,
    )
    for _ in range(3):
        y = y * (1.5 - 0.5 * t * y * y)
    return y


def _cosine_body(
    z_hbm, src_hbm, dst_hbm, out_hbm,
    si_v, di_v, o_v, a0, b0, a1, b1, srd, nloc_v, nrm_v, shared_nrm,
    sem_a0, sem_b0, sem_a1, sem_b1,
):
    cid = lax.axis_index("c")
    sid = lax.axis_index("s")
    wid = sid * jnp.int32(NUM_CORES) + cid
    lanes = lax.iota(jnp.int32, LANES)
    rows = lanes
    lastc = jnp.full((LANES,), LANES - 1, jnp.int32)

    nbase = sid * jnp.int32(NODES_PER_TILE)

    def norm_chunk(c, carry):
        pltpu.sync_copy(z_hbm.at[pl.ds(nbase + c * jnp.int32(CHUNK), CHUNK)], a0)

        def norm_group(g, carry2):
            for j in range(LANES):
                e = g * jnp.int32(LANES) + jnp.int32(j)
                va = [a0[e, pl.ds(LANES * k, LANES)] for k in range(NSUB)]
                qa = _tree_sum([va[k] * va[k] for k in range(NSUB)])
                srd[j, pl.ds(0, LANES)] = plsc.cumsum(qa)
            tot = plsc.load_gather(srd, [rows, lastc])
            nloc_v[pl.ds(c * jnp.int32(CHUNK) + g * jnp.int32(LANES), LANES)] = tot
            return carry2

        lax.fori_loop(jnp.int32(0), jnp.int32(CHUNK // LANES), norm_group, jnp.int32(0))
        return carry

    lax.fori_loop(jnp.int32(0), jnp.int32(NORM_CHUNKS), norm_chunk, jnp.int32(0))
    pltpu.sync_copy(nloc_v, shared_nrm.at[pl.ds(nbase, NODES_PER_TILE)])
    plsc.subcore_barrier()
    pltpu.sync_copy(shared_nrm, nrm_v)

    base_w = wid * jnp.int32(EDGES_PER_WORKER)
    pltpu.sync_copy(src_hbm.at[pl.ds(base_w, EDGES_PER_WORKER)], si_v)
    pltpu.sync_copy(dst_hbm.at[pl.ds(base_w, EDGES_PER_WORKER)], di_v)

    abuf = (a0, a1)
    bbuf = (b0, b1)
    asem = (sem_a0, sem_a1)
    bsem = (sem_b0, sem_b1)

    def issue(ci, p):
        off = ci * jnp.int32(CHUNK)
        pltpu.async_copy(z_hbm.at[si_v.at[pl.ds(off, CHUNK)]], abuf[p], asem[p])
        pltpu.async_copy(z_hbm.at[di_v.at[pl.ds(off, CHUNK)]], bbuf[p], bsem[p])

    def wait(p):
        pltpu.make_async_copy(z_hbm.at[si_v.at[pl.ds(0, CHUNK)]], abuf[p], asem[p]).wait()
        pltpu.make_async_copy(z_hbm.at[di_v.at[pl.ds(0, CHUNK)]], bbuf[p], bsem[p]).wait()

    def compute(ci, p):
        av, bv = abuf[p], bbuf[p]
        obase = ci * jnp.int32(CHUNK)

        def group(g, carry):
            goff = obase + g * jnp.int32(LANES)
            e_idx = g * jnp.int32(LANES) + lanes
            sidx = si_v[pl.ds(goff, LANES)]
            didx = di_v[pl.ds(goff, LANES)]
            na2 = plsc.load_gather(nrm_v, [sidx])
            nb2 = plsc.load_gather(nrm_v, [didx])
            zeros = jnp.zeros((LANES,), jnp.float32)

            def fstep(f8, acc):
                base = lanes + f8 * jnp.int32(8)
                acc = list(acc)
                for k in range(8):
                    fi0 = (base + jnp.int32(k)) & jnp.int32(D_FEAT - 1)
                    fi1 = fi0 ^ jnp.int32(D_FEAT // 2)
                    va0 = plsc.load_gather(av, [e_idx, fi0])
                    vb0 = plsc.load_gather(bv, [e_idx, fi0])
                    va1 = plsc.load_gather(av, [e_idx, fi1])
                    vb1 = plsc.load_gather(bv, [e_idx, fi1])
                    acc[k % 4] = acc[k % 4] + va0 * vb0
                    acc[4 + k % 4] = acc[4 + k % 4] + va1 * vb1
                return tuple(acc)

            acc = lax.fori_loop(
                jnp.int32(0), jnp.int32(D_FEAT // 16), fstep, (zeros,) * 8
            )
            dot = _tree_sum(list(acc))
            t = na2 * nb2
            s = t * _vec_rsqrt(t)
            o_v[pl.ds(goff, LANES)] = dot / jnp.maximum(s, EPS)
            return carry

        lax.fori_loop(jnp.int32(0), jnp.int32(CHUNK // LANES), group, jnp.int32(0))

    last = jnp.int32(NUM_CHUNKS - 1)
    issue(jnp.int32(0), 0)

    def pair(pi, carry):
        c0 = pi * jnp.int32(2)
        c1 = jnp.minimum(c0 + 1, last)
        c2 = jnp.minimum(c0 + 2, last)
        issue(c1, 1)
        wait(0)
        compute(c0, 0)
        issue(c2, 0)
        wait(1)
        compute(c1, 1)
        return carry

    lax.fori_loop(jnp.int32(0), jnp.int32(NUM_PAIRS), pair, jnp.int32(0))
    wait(0)
    pltpu.sync_copy(o_v, out_hbm.at[pl.ds(base_w, EDGES_PER_WORKER)])


_cosine_sc = functools.partial(
    pl.kernel,
    out_type=jax.ShapeDtypeStruct((N_EDGES,), jnp.float32),
    mesh=plsc.VectorSubcoreMesh(core_axis_name="c", subcore_axis_name="s"),
    compiler_params=pltpu.CompilerParams(needs_layout_passes=False),
    scratch_types=[
        pltpu.VMEM((EDGES_PER_WORKER,), jnp.int32),
        pltpu.VMEM((EDGES_PER_WORKER,), jnp.int32),
        pltpu.VMEM((EDGES_PER_WORKER,), jnp.float32),
        pltpu.VMEM((CHUNK, D_FEAT), jnp.float32),
        pltpu.VMEM((CHUNK, D_FEAT), jnp.float32),
        pltpu.VMEM((CHUNK, D_FEAT), jnp.float32),
        pltpu.VMEM((CHUNK, D_FEAT), jnp.float32),
        pltpu.VMEM((LANES, LANES + 1), jnp.float32),
        pltpu.VMEM((NODES_PER_TILE,), jnp.float32),
        pltpu.VMEM((N_PAD,), jnp.float32),
        pltpu.VMEM_SHARED((N_PAD,), jnp.float32),
        pltpu.SemaphoreType.DMA,
        pltpu.SemaphoreType.DMA,
        pltpu.SemaphoreType.DMA,
        pltpu.SemaphoreType.DMA,
    ],
)(_cosine_body)


def kernel(z, edge_index):
    z_pad = jnp.concatenate(
        [z, jnp.zeros((N_PAD - N_NODES, D_FEAT), jnp.float32)], axis=0
    )
    src = edge_index[0].astype(jnp.int32)
    dst = edge_index[1].astype(jnp.int32)
    return _cosine_sc(z_pad, src, dst)
